# Initial kernel scaffold; baseline (speedup 1.0000x reference)
import jax
import jax.numpy as jnp
from jax.experimental import pallas as pl


def kernel(x, pos, batch, x_skip, pos_skip, batch_skip, W_att, W_nn, b_nn, gamma, beta, k):
    raise NotImplementedError("write your pallas kernel here")


# speedup vs baseline: 6.8132x; 6.8132x over previous
import functools

import jax
import jax.numpy as jnp
from jax import lax
from jax.experimental import pallas as pl
from jax.experimental.pallas import tpu as pltpu
from jax.experimental.pallas import tpu_sc as plsc

N_X = 4096
N_Y = 16384
C = 256
K = 3
BQ = 256
NBLK = N_Y // BQ

def _att_body(wT_ref, x_ref, att_ref):
    att_ref[...] = lax.dot_general(
        wT_ref[...].astype(jnp.bfloat16), x_ref[...].astype(jnp.bfloat16),
        (((1,), (1,)), ((), ())),
        preferred_element_type=jnp.float32)


def _att_call(W_att, x):
    wT = W_att.reshape(1, C)
    return pl.pallas_call(
        _att_body,
        out_shape=jax.ShapeDtypeStruct((1, N_X), jnp.float32),
    )(wT, x)


def _knn_body(ps_ref, posT_ref, att_ref, idx_ref, coef_ref):
    ps = ps_ref[...]
    px = posT_ref[0:1, :]
    py = posT_ref[1:2, :]
    pz = posT_ref[2:3, :]
    qx = ps[:, 0:1]
    qy = ps[:, 1:2]
    qz = ps[:, 2:3]
    knorm = px * px + py * py + pz * pz
    qnorm = qx * qx + qy * qy + qz * qz
    pxb = px.astype(jnp.bfloat16).astype(jnp.float32)
    pyb = py.astype(jnp.bfloat16).astype(jnp.float32)
    pzb = pz.astype(jnp.bfloat16).astype(jnp.float32)
    qxb = qx.astype(jnp.bfloat16).astype(jnp.float32)
    qyb = qy.astype(jnp.bfloat16).astype(jnp.float32)
    qzb = qz.astype(jnp.bfloat16).astype(jnp.float32)
    cross = qxb * pxb + qyb * pyb + qzb * pzb
    d2 = qnorm + knorm - 2.0 * cross

    iota = lax.broadcasted_iota(jnp.int32, (BQ, N_X), 1)
    att_row = att_ref[...]
    inf = jnp.float32(jnp.inf)

    idxs, atts, ws = [], [], []
    d2w = d2
    for _ in range(K):
        m = jnp.min(d2w, axis=1, keepdims=True)
        cand = jnp.where(d2w == m, iota, N_X)
        idxj = jnp.min(cand, axis=1, keepdims=True)
        onehot = iota == idxj
        d2w = jnp.where(onehot, inf, d2w)
        attj = jnp.sum(jnp.where(onehot, att_row, 0.0), axis=1, keepdims=True)
        gx = jnp.sum(jnp.where(onehot, px, 0.0), axis=1, keepdims=True)
        gy = jnp.sum(jnp.where(onehot, py, 0.0), axis=1, keepdims=True)
        gz = jnp.sum(jnp.where(onehot, pz, 0.0), axis=1, keepdims=True)
        dx = gx - qx
        dy = gy - qy
        dz = gz - qz
        sqd = dx * dx + dy * dy + dz * dz
        wj = 1.0 / jnp.maximum(sqd, 1e-16)
        idxs.append(idxj)
        atts.append(attj)
        ws.append(wj)

    amax = jnp.maximum(jnp.maximum(atts[0], atts[1]), atts[2])
    es = [jnp.exp(a - amax) for a in atts]
    s = es[0] + es[1] + es[2]
    den = ws[0] + ws[1] + ws[2]
    sden = s * den
    coefs = [e * w / sden for e, w in zip(es, ws)]

    idx_ref[...] = jnp.concatenate(idxs, axis=1)
    coef_ref[...] = jnp.concatenate(coefs, axis=1)


def _knn_call(pos_skip, pos_T, att):
    return pl.pallas_call(
        _knn_body,
        grid=(NBLK,),
        in_specs=[
            pl.BlockSpec((BQ, 3), lambda i: (i, 0)),
            pl.BlockSpec((3, N_X), lambda i: (0, 0)),
            pl.BlockSpec((1, N_X), lambda i: (0, 0)),
        ],
        out_specs=[
            pl.BlockSpec((BQ, K), lambda i: (i, 0)),
            pl.BlockSpec((BQ, K), lambda i: (i, 0)),
        ],
        out_shape=[
            jax.ShapeDtypeStruct((N_Y, K), jnp.int32),
            jax.ShapeDtypeStruct((N_Y, K), jnp.float32),
        ],
    )(pos_skip, pos_T, att)


_ROWS = N_Y * K
_NW = 32
_RPW = _ROWS // _NW
_CH = 128
_NCHUNK = _RPW // _CH


def _sc_gather(x, idx_flat):
    mesh = plsc.VectorSubcoreMesh(core_axis_name="c", subcore_axis_name="s")

    @functools.partial(
        pl.kernel,
        mesh=mesh,
        out_type=jax.ShapeDtypeStruct((_ROWS, C), jnp.float32),
        scratch_types=[
            pltpu.VMEM((2, _CH), jnp.int32),
            pltpu.VMEM((2, _CH, C), jnp.float32),
            pltpu.SemaphoreType.DMA,
            pltpu.SemaphoreType.DMA,
            pltpu.SemaphoreType.DMA,
            pltpu.SemaphoreType.DMA,
        ],
    )
    def k(x_hbm, idx_hbm, out_hbm, idx_v, rows_v, g0, g1, s0, s1):
        wid = lax.axis_index("s") * 2 + lax.axis_index("c")
        base = wid * _RPW
        gsem = [g0, g1]
        ssem = [s0, s1]
        gath = [None, None]
        scat = [None, None]
        for c in range(_NCHUNK):
            p = c % 2
            if scat[p] is not None:
                scat[p].wait()
            pltpu.sync_copy(idx_hbm.at[pl.ds(base + c * _CH, _CH)],
                            idx_v.at[p])
            gath[p] = pltpu.async_copy(x_hbm.at[idx_v.at[p]], rows_v.at[p],
                                       gsem[p])
            q = 1 - p
            if gath[q] is not None:
                gath[q].wait()
                gath[q] = None
                scat[q] = pltpu.async_copy(
                    rows_v.at[q],
                    out_hbm.at[pl.ds(base + (c - 1) * _CH, _CH)], ssem[q])
        pl_last = (_NCHUNK - 1) % 2
        gath[pl_last].wait()
        scat[pl_last] = pltpu.async_copy(
            rows_v.at[pl_last],
            out_hbm.at[pl.ds(base + (_NCHUNK - 1) * _CH, _CH)],
            ssem[pl_last])
        scat[0].wait()
        scat[1].wait()

    return k(x, idx_flat)


def _comb_body(g_ref, coef_ref, xs_ref, w1_ref, w2_ref, b_ref,
               z_ref, s1_ref, s2_ref):
    i = pl.program_id(0)
    g = g_ref[...]
    cf = coef_ref[...]
    y = (g[:, 0:C] * cf[:, 0:1]
         + g[:, C:2 * C] * cf[:, 1:2]
         + g[:, 2 * C:3 * C] * cf[:, 2:3])
    z = (jnp.dot(y.astype(jnp.bfloat16), w1_ref[...].astype(jnp.bfloat16),
                 preferred_element_type=jnp.float32)
         + jnp.dot(xs_ref[...].astype(jnp.bfloat16),
                   w2_ref[...].astype(jnp.bfloat16),
                   preferred_element_type=jnp.float32)
         + b_ref[...])
    z_ref[...] = z
    ps1 = jnp.sum(z, axis=0, keepdims=True)
    ps2 = jnp.sum(z * z, axis=0, keepdims=True)

    @pl.when(i == 0)
    def _():
        s1_ref[...] = ps1
        s2_ref[...] = ps2

    @pl.when(i > 0)
    def _():
        s1_ref[...] += ps1
        s2_ref[...] += ps2


def _comb_call(g, coef, x_skip, W1, W2, b):
    return pl.pallas_call(
        _comb_body,
        grid=(NBLK,),
        in_specs=[
            pl.BlockSpec((BQ, K * C), lambda i: (i, 0)),
            pl.BlockSpec((BQ, K), lambda i: (i, 0)),
            pl.BlockSpec((BQ, C), lambda i: (i, 0)),
            pl.BlockSpec((C, C), lambda i: (0, 0)),
            pl.BlockSpec((C, C), lambda i: (0, 0)),
            pl.BlockSpec((1, C), lambda i: (0, 0)),
        ],
        out_specs=[
            pl.BlockSpec((BQ, C), lambda i: (i, 0)),
            pl.BlockSpec((1, C), lambda i: (0, 0)),
            pl.BlockSpec((1, C), lambda i: (0, 0)),
        ],
        out_shape=[
            jax.ShapeDtypeStruct((N_Y, C), jnp.float32),
            jax.ShapeDtypeStruct((1, C), jnp.float32),
            jax.ShapeDtypeStruct((1, C), jnp.float32),
        ],
    )(g, coef, x_skip, W1, W2, b)


def _norm_body(z_ref, s1_ref, s2_ref, gm_ref, bt_ref, o_ref):
    n = jnp.float32(N_Y)
    mean = s1_ref[...] / n
    var = s2_ref[...] / n - mean * mean
    inv = 1.0 / jnp.sqrt(var + 1e-6)
    zz = (z_ref[...] - mean) * inv * gm_ref[...] + bt_ref[...]
    o_ref[...] = jnp.where(zz > 0, zz, 0.2 * zz)


def _norm_call(z, s1, s2, gamma, beta):
    return pl.pallas_call(
        _norm_body,
        grid=(NBLK,),
        in_specs=[
            pl.BlockSpec((BQ, C), lambda i: (i, 0)),
            pl.BlockSpec((1, C), lambda i: (0, 0)),
            pl.BlockSpec((1, C), lambda i: (0, 0)),
            pl.BlockSpec((1, C), lambda i: (0, 0)),
            pl.BlockSpec((1, C), lambda i: (0, 0)),
        ],
        out_specs=pl.BlockSpec((BQ, C), lambda i: (i, 0)),
        out_shape=jax.ShapeDtypeStruct((N_Y, C), jnp.float32),
    )(z, s1, s2, gamma, beta)


def _gather_rows(x, idx_flat):
    return _sc_gather(x, idx_flat)


def kernel(x, pos, batch, x_skip, pos_skip, batch_skip, W_att, W_nn, b_nn,
           gamma, beta, k):
    pos_T = pos.T
    att = _att_call(W_att, x)
    idx, coef = _knn_call(pos_skip, pos_T, att)
    g = _gather_rows(x, idx.reshape(-1))
    g = g.reshape(N_Y, K * C)
    W1 = W_nn[:C]
    W2 = W_nn[C:]
    z, s1, s2 = _comb_call(g, coef, x_skip, W1, W2, b_nn.reshape(1, C))
    out = _norm_call(z, s1, s2, gamma.reshape(1, C), beta.reshape(1, C))
    return (out, pos_skip, batch_skip)


# speedup vs baseline: 11.8297x; 1.7363x over previous
import functools

import jax
import jax.numpy as jnp
from jax import lax
from jax.experimental import pallas as pl
from jax.experimental.pallas import tpu as pltpu
from jax.experimental.pallas import tpu_sc as plsc

N_X = 4096
N_Y = 16384
C = 256
K = 3
TD = 4
BQ = 256
NBLK = N_Y // BQ

def _att_body(wT_ref, x_ref, att_ref):
    att_ref[...] = lax.dot_general(
        wT_ref[...].astype(jnp.bfloat16), x_ref[...].astype(jnp.bfloat16),
        (((1,), (1,)), ((), ())),
        preferred_element_type=jnp.float32)


def _att_call(W_att, x):
    wT = W_att.reshape(1, C)
    return pl.pallas_call(
        _att_body,
        out_shape=jax.ShapeDtypeStruct((1, N_X), jnp.float32),
    )(wT, x)


def _knn_body(ps_ref, posT_ref, idx_ref):
    ps = ps_ref[...]
    px = posT_ref[0:1, :]
    py = posT_ref[1:2, :]
    pz = posT_ref[2:3, :]
    qx = ps[:, 0:1]
    qy = ps[:, 1:2]
    qz = ps[:, 2:3]
    knorm = px * px + py * py + pz * pz
    qnorm = qx * qx + qy * qy + qz * qz
    pxb = px.astype(jnp.bfloat16).astype(jnp.float32)
    pyb = py.astype(jnp.bfloat16).astype(jnp.float32)
    pzb = pz.astype(jnp.bfloat16).astype(jnp.float32)
    qxb = qx.astype(jnp.bfloat16).astype(jnp.float32)
    qyb = qy.astype(jnp.bfloat16).astype(jnp.float32)
    qzb = qz.astype(jnp.bfloat16).astype(jnp.float32)
    cross = qxb * pxb + qyb * pyb + qzb * pzb
    d2 = qnorm + knorm - 2.0 * cross

    iota = lax.broadcasted_iota(jnp.int32, (BQ, N_X), 1)
    inf = jnp.float32(jnp.inf)

    idxs = []
    d2w = d2
    for j in range(K):
        m = jnp.min(d2w, axis=1, keepdims=True)
        cand = jnp.where(d2w == m, iota, N_X)
        idxj = jnp.min(cand, axis=1, keepdims=True)
        if j < K - 1:
            d2w = jnp.where(iota == idxj, inf, d2w)
        idxs.append(idxj)

    idx_ref[...] = jnp.concatenate(idxs, axis=1)


def _knn_call(pos_skip, pos_T):
    return pl.pallas_call(
        _knn_body,
        grid=(NBLK,),
        in_specs=[
            pl.BlockSpec((BQ, 3), lambda i: (i, 0)),
            pl.BlockSpec((3, N_X), lambda i: (0, 0)),
        ],
        out_specs=pl.BlockSpec((BQ, K), lambda i: (i, 0)),
        out_shape=jax.ShapeDtypeStruct((N_Y, K), jnp.int32),
    )(pos_skip, pos_T)


_NW = 32
_QPW = N_Y // _NW
_CQ = 32
_NCH = _QPW // _CQ
_CR = _CQ * K


def _sc_combine(x, t_flat, idx_flat, ps_flat):
    mesh = plsc.VectorSubcoreMesh(core_axis_name="c", subcore_axis_name="s")

    @functools.partial(
        pl.kernel,
        mesh=mesh,
        compiler_params=pltpu.CompilerParams(needs_layout_passes=False),
        out_type=jax.ShapeDtypeStruct((N_Y, C), jnp.float32),
        scratch_types=[
            pltpu.VMEM((_QPW * K,), jnp.int32),
            pltpu.VMEM((_QPW * K,), jnp.float32),
            pltpu.VMEM((N_X * TD,), jnp.float32),
            pltpu.VMEM((_QPW * K,), jnp.float32),
            pltpu.VMEM((_CR, C), jnp.float32),
            pltpu.VMEM((_CR, C), jnp.float32),
            pltpu.VMEM((_CQ, C), jnp.float32),
            pltpu.VMEM((_CQ, C), jnp.float32),
            pltpu.SemaphoreType.DMA,
            pltpu.SemaphoreType.DMA,
            pltpu.SemaphoreType.DMA,
            pltpu.SemaphoreType.DMA,
        ],
    )
    def k(x_hbm, t_hbm, idx_hbm, ps_hbm, y_hbm,
          idx_w, ps_w, t_all, coef_w, rows0, rows1, y0, y1,
          gx0, gx1, sy0, sy1):
        wid = lax.axis_index("s") * 2 + lax.axis_index("c")
        qbase0 = wid * _QPW
        rows_b = [rows0, rows1]
        y_b = [y0, y1]
        gxs = [gx0, gx1]
        sys_ = [sy0, sy1]
        gx = [None, None]
        sc = [None, None]
        iota16 = lax.broadcasted_iota(jnp.int32, (16,), 0)

        pltpu.sync_copy(idx_hbm.at[pl.ds(qbase0 * K, _QPW * K)], idx_w)
        pltpu.sync_copy(ps_hbm.at[pl.ds(qbase0 * K, _QPW * K)], ps_w)
        pltpu.sync_copy(t_hbm, t_all)

        for cq in range(2):
            gx[cq] = pltpu.async_copy(
                x_hbm.at[idx_w.at[pl.ds(cq * _CR, _CR)]], rows_b[cq], gxs[cq])

        for g in range(_QPW // 16):
            qi = iota16 + 16 * g
            r0 = qi * K
            psx = plsc.load_gather(ps_w, [r0])
            psy = plsc.load_gather(ps_w, [r0 + 1])
            psz = plsc.load_gather(ps_w, [r0 + 2])
            ws = []
            es = []
            for j in range(K):
                gi = plsc.load_gather(idx_w, [r0 + j]) * TD
                tx = plsc.load_gather(t_all, [gi])
                ty = plsc.load_gather(t_all, [gi + 1])
                tz = plsc.load_gather(t_all, [gi + 2])
                ta = plsc.load_gather(t_all, [gi + 3])
                dx = tx - psx
                dy = ty - psy
                dz = tz - psz
                sqd = dx * dx + dy * dy + dz * dz
                ws.append(1.0 / jnp.maximum(sqd, 1e-16))
                es.append(ta)
            amax = jnp.maximum(jnp.maximum(es[0], es[1]), es[2])
            es = [jnp.exp(a - amax) for a in es]
            s = es[0] + es[1] + es[2]
            den = ws[0] + ws[1] + ws[2]
            sden = s * den
            for j in range(K):
                plsc.store_scatter(coef_w, [r0 + j], es[j] * ws[j] / sden)

        def combine(p, cq):
            rows_v = rows_b[p]
            y_v = y_b[p]

            def body(q, _):
                qv = jnp.full((16,), q, jnp.int32)
                cb = (cq * _CQ + q) * K
                c0 = plsc.load_gather(coef_w, [jnp.full((16,), cb, jnp.int32)])
                c1 = plsc.load_gather(coef_w, [jnp.full((16,), cb + 1, jnp.int32)])
                c2 = plsc.load_gather(coef_w, [jnp.full((16,), cb + 2, jnp.int32)])
                rq = qv * K
                for v in range(C // 16):
                    col = iota16 + 16 * v
                    a0 = plsc.load_gather(rows_v, [rq, col])
                    a1 = plsc.load_gather(rows_v, [rq + 1, col])
                    a2 = plsc.load_gather(rows_v, [rq + 2, col])
                    yv = c0 * a0 + c1 * a1 + c2 * a2
                    plsc.store_scatter(y_v, [qv, col], yv)
                return 0
            lax.fori_loop(0, _CQ, body, 0, unroll=False)

        for cq in range(_NCH):
            p = cq % 2
            gx[p].wait()
            if sc[p] is not None:
                sc[p].wait()
            combine(p, cq)
            sc[p] = pltpu.async_copy(
                y_b[p], y_hbm.at[pl.ds(qbase0 + cq * _CQ, _CQ)], sys_[p])
            if cq + 2 < _NCH:
                gx[p] = pltpu.async_copy(
                    x_hbm.at[idx_w.at[pl.ds((cq + 2) * _CR, _CR)]],
                    rows_b[p], gxs[p])
        sc[0].wait()
        sc[1].wait()

    return k(x, t_flat, idx_flat, ps_flat)


def _comb_body(y_ref, xs_ref, w1_ref, w2_ref, b_ref, z_ref, s1_ref, s2_ref):
    i = pl.program_id(0)
    z = (jnp.dot(y_ref[...].astype(jnp.bfloat16), w1_ref[...].astype(jnp.bfloat16),
                 preferred_element_type=jnp.float32)
         + jnp.dot(xs_ref[...].astype(jnp.bfloat16),
                   w2_ref[...].astype(jnp.bfloat16),
                   preferred_element_type=jnp.float32)
         + b_ref[...])
    z_ref[...] = z
    ps1 = jnp.sum(z, axis=0, keepdims=True)
    ps2 = jnp.sum(z * z, axis=0, keepdims=True)

    @pl.when(i == 0)
    def _():
        s1_ref[...] = ps1
        s2_ref[...] = ps2

    @pl.when(i > 0)
    def _():
        s1_ref[...] += ps1
        s2_ref[...] += ps2


def _comb_call(y, x_skip, W1, W2, b):
    return pl.pallas_call(
        _comb_body,
        grid=(NBLK,),
        in_specs=[
            pl.BlockSpec((BQ, C), lambda i: (i, 0)),
            pl.BlockSpec((BQ, C), lambda i: (i, 0)),
            pl.BlockSpec((C, C), lambda i: (0, 0)),
            pl.BlockSpec((C, C), lambda i: (0, 0)),
            pl.BlockSpec((1, C), lambda i: (0, 0)),
        ],
        out_specs=[
            pl.BlockSpec((BQ, C), lambda i: (i, 0)),
            pl.BlockSpec((1, C), lambda i: (0, 0)),
            pl.BlockSpec((1, C), lambda i: (0, 0)),
        ],
        out_shape=[
            jax.ShapeDtypeStruct((N_Y, C), jnp.float32),
            jax.ShapeDtypeStruct((1, C), jnp.float32),
            jax.ShapeDtypeStruct((1, C), jnp.float32),
        ],
    )(y, x_skip, W1, W2, b)


def _norm_body(z_ref, s1_ref, s2_ref, gm_ref, bt_ref, o_ref):
    n = jnp.float32(N_Y)
    mean = s1_ref[...] / n
    var = s2_ref[...] / n - mean * mean
    inv = 1.0 / jnp.sqrt(var + 1e-6)
    zz = (z_ref[...] - mean) * inv * gm_ref[...] + bt_ref[...]
    o_ref[...] = jnp.where(zz > 0, zz, 0.2 * zz)


def _norm_call(z, s1, s2, gamma, beta):
    return pl.pallas_call(
        _norm_body,
        grid=(NBLK,),
        in_specs=[
            pl.BlockSpec((BQ, C), lambda i: (i, 0)),
            pl.BlockSpec((1, C), lambda i: (0, 0)),
            pl.BlockSpec((1, C), lambda i: (0, 0)),
            pl.BlockSpec((1, C), lambda i: (0, 0)),
            pl.BlockSpec((1, C), lambda i: (0, 0)),
        ],
        out_specs=pl.BlockSpec((BQ, C), lambda i: (i, 0)),
        out_shape=jax.ShapeDtypeStruct((N_Y, C), jnp.float32),
    )(z, s1, s2, gamma, beta)


def _interp_y(x, t, idx, pos_skip):
    return _sc_combine(x, t.reshape(-1), idx.reshape(-1), pos_skip.reshape(-1))


def kernel(x, pos, batch, x_skip, pos_skip, batch_skip, W_att, W_nn, b_nn,
           gamma, beta, k):
    pos_T = pos.T
    att = _att_call(W_att, x)
    idx = _knn_call(pos_skip, pos_T)
    t = jnp.concatenate([pos, att.reshape(N_X, 1)], axis=1)
    y = _interp_y(x, t, idx, pos_skip)
    W1 = W_nn[:C]
    W2 = W_nn[C:]
    z, s1, s2 = _comb_call(y, x_skip, W1, W2, b_nn.reshape(1, C))
    out = _norm_call(z, s1, s2, gamma.reshape(1, C), beta.reshape(1, C))
    return (out, pos_skip, batch_skip)
